# baseline (speedup 1.0000x reference)
import functools

import jax
import jax.numpy as jnp
from jax import lax
from jax.experimental import pallas as pl
from jax.experimental.pallas import tpu as pltpu
from jax.experimental.pallas import tpu_sc as plsc

_B, _S, _D = 4, 8192, 1024
_NW = 32
_ROWS_PER_W = (_B * _S) // _NW
_C = 16
_NCHUNK = _ROWS_PER_W // _C
_NV = _D // 16


def _sc_body(in_hbm, pos_hbm, seg_hbm, out_hbm, inb, posb, segb, sem):
    cid = lax.axis_index("c")
    sid = lax.axis_index("s")
    wid = sid * 2 + cid
    b = wid // 8
    s_base = (wid % 8) * _ROWS_PER_W

    pltpu.sync_copy(seg_hbm, segb)

    def chunk(it, carry):
        s0 = s_base + it * _C
        pltpu.async_copy(in_hbm.at[b, pl.ds(s0, _C), :], inb, sem).wait()
        pltpu.async_copy(pos_hbm.at[pl.ds(s0, _C), :], posb, sem).wait()

        ws = [
            jnp.full((16,), jnp.where(s0 + r > _S // 2, 1.0, 0.0), jnp.float32)
            for r in range(_C)
        ]

        def vloop(v, carry2):
            dsv = pl.ds(v * 16, 16)
            s0v = segb[0, dsv]
            dv = segb[1, dsv] - s0v
            for r in range(_C):
                inb[r, dsv] = inb[r, dsv] + posb[r, dsv] + (s0v + ws[r] * dv)
            return carry2

        lax.fori_loop(0, _NV, vloop, 0, unroll=False)
        pltpu.sync_copy(inb, out_hbm.at[b, pl.ds(s0, _C), :])
        return carry

    lax.fori_loop(0, _NCHUNK, chunk, 0, unroll=False)


def kernel(input_embedding, position_table, segment_table):
    mesh = plsc.VectorSubcoreMesh(core_axis_name="c", subcore_axis_name="s")
    fn = pl.kernel(
        _sc_body,
        out_type=jax.ShapeDtypeStruct((_B, _S, _D), jnp.float32),
        mesh=mesh,
        scratch_types=[
            pltpu.VMEM((_C, _D), jnp.float32),
            pltpu.VMEM((_C, _D), jnp.float32),
            pltpu.VMEM((2, _D), jnp.float32),
            pltpu.SemaphoreType.DMA,
        ],
    )
    return fn(input_embedding, position_table[:_S], segment_table)


# speedup vs baseline: 1.0194x; 1.0194x over previous
import functools

import jax
import jax.numpy as jnp
from jax import lax
from jax.experimental import pallas as pl
from jax.experimental.pallas import tpu as pltpu
from jax.experimental.pallas import tpu_sc as plsc

_B, _S, _D = 4, 8192, 1024
_NW = 32
_ROWS_PER_W = (_B * _S) // _NW
_C = 32
_NCHUNK = _ROWS_PER_W // _C
_NV = _D // 16


def _sc_body(in_hbm, pos_hbm, seg_hbm, out_hbm, inb, posb, segb, sem):
    cid = lax.axis_index("c")
    sid = lax.axis_index("s")
    wid = sid * 2 + cid
    b = wid // 8
    s_base = (wid % 8) * _ROWS_PER_W

    pltpu.sync_copy(seg_hbm, segb)

    def chunk(it, carry):
        s0 = s_base + it * _C
        cp_in = pltpu.async_copy(in_hbm.at[b, pl.ds(s0, _C), :], inb, sem)
        cp_pos = pltpu.async_copy(pos_hbm.at[pl.ds(s0, _C), :], posb, sem)
        cp_in.wait()
        cp_pos.wait()

        ws = [
            jnp.full((16,), jnp.where(s0 + r > _S // 2, 1.0, 0.0), jnp.float32)
            for r in range(_C)
        ]

        def vloop(v, carry2):
            dsv = pl.ds(v * 16, 16)
            s0v = segb[0, dsv]
            dv = segb[1, dsv] - s0v
            for r in range(_C):
                inb[r, dsv] = inb[r, dsv] + posb[r, dsv] + (s0v + ws[r] * dv)
            return carry2

        lax.fori_loop(0, _NV, vloop, 0, unroll=False)
        pltpu.sync_copy(inb, out_hbm.at[b, pl.ds(s0, _C), :])
        return carry

    lax.fori_loop(0, _NCHUNK, chunk, 0, unroll=False)


def kernel(input_embedding, position_table, segment_table):
    mesh = plsc.VectorSubcoreMesh(core_axis_name="c", subcore_axis_name="s")
    fn = pl.kernel(
        _sc_body,
        out_type=jax.ShapeDtypeStruct((_B, _S, _D), jnp.float32),
        mesh=mesh,
        scratch_types=[
            pltpu.VMEM((_C, _D), jnp.float32),
            pltpu.VMEM((_C, _D), jnp.float32),
            pltpu.VMEM((2, _D), jnp.float32),
            pltpu.SemaphoreType.DMA,
        ],
    )
    return fn(input_embedding, position_table[:_S], segment_table)


# speedup vs baseline: 5.5421x; 5.4364x over previous
import jax
import jax.numpy as jnp
from jax.experimental import pallas as pl

_SEQ_BLOCK = 2048


def _make_body(half_len):
    def _body(inp_ref, pos_ref, seg_ref, out_ref):
        base = pl.program_id(0) * _SEQ_BLOCK
        idx = base + jax.lax.broadcasted_iota(jnp.int32, (_SEQ_BLOCK, 1), 0)
        mask = idx > half_len
        seg = jnp.where(mask, seg_ref[1, :][None, :], seg_ref[0, :][None, :])
        out_ref[...] = inp_ref[...] + (pos_ref[...] + seg)[None]

    return _body


def kernel(input_embedding, position_table, segment_table):
    B, S, D = input_embedding.shape
    n_seq = S // _SEQ_BLOCK
    return pl.pallas_call(
        _make_body(S // 2),
        grid=(n_seq, B),
        in_specs=[
            pl.BlockSpec((1, _SEQ_BLOCK, D), lambda i, j: (j, i, 0)),
            pl.BlockSpec((_SEQ_BLOCK, D), lambda i, j: (i, 0)),
            pl.BlockSpec(segment_table.shape, lambda i, j: (0, 0)),
        ],
        out_specs=pl.BlockSpec((1, _SEQ_BLOCK, D), lambda i, j: (j, i, 0)),
        out_shape=jax.ShapeDtypeStruct((B, S, D), input_embedding.dtype),
    )(input_embedding, position_table[:S], segment_table)
